# baseline (speedup 1.0000x reference)
import functools
import jax
import jax.numpy as jnp
from jax.experimental import pallas as pl
from jax.experimental.pallas import tpu as pltpu

K = 3
BM = 512


def _knn_l1_kernel(pT_ref, q_ref, x_ref, y_ref, Wx_ref, Wy_ref, b1_ref,
                   h1_ref, s1_ref, ss1_ref, *, n_src):
    b = pl.program_id(0)
    j = pl.program_id(1)

    q_blk = q_ref[0]
    pT = pT_ref[0]

    qp = jax.lax.dot_general(q_blk, pT,
                             dimension_numbers=(((1,), (0,)), ((), ())),
                             precision=jax.lax.Precision.HIGHEST,
                             preferred_element_type=jnp.float32)
    pp = (pT[0:1, :] * pT[0:1, :] + pT[1:2, :] * pT[1:2, :]
          + pT[2:3, :] * pT[2:3, :])
    qq = jnp.sum(q_blk * q_blk, axis=1, keepdims=True)
    d = pp - 2.0 * qp

    inf = jnp.float32(jnp.inf)
    t1 = jnp.min(d, axis=1, keepdims=True)
    dm1 = jnp.where(d == t1, inf, d)
    t2 = jnp.min(dm1, axis=1, keepdims=True)
    dm2 = jnp.where(dm1 == t2, inf, dm1)
    t3 = jnp.min(dm2, axis=1, keepdims=True)

    w1 = 1.0 / jnp.maximum(t1 + qq, 1e-10)
    w2 = 1.0 / jnp.maximum(t2 + qq, 1e-10)
    w3 = 1.0 / jnp.maximum(t3 + qq, 1e-10)
    wsum = w1 + w2 + w3
    wn1 = w1 / wsum
    wn2 = w2 / wsum
    wn3 = w3 / wsum

    ST = jnp.where(d == t1, wn1,
                   jnp.where(dm1 == t2, wn2,
                             jnp.where(dm2 == t3, wn3, 0.0))
                   ).astype(jnp.bfloat16)

    xi = jax.lax.dot_general(x_ref[0], ST,
                             dimension_numbers=(((1,), (1,)), ((), ())),
                             preferred_element_type=jnp.float32)

    h1 = (jax.lax.dot_general(Wx_ref[...], xi.astype(jnp.bfloat16),
                              dimension_numbers=(((1,), (0,)), ((), ())),
                              preferred_element_type=jnp.float32)
          + jax.lax.dot_general(Wy_ref[...], y_ref[0],
                                dimension_numbers=(((1,), (0,)), ((), ())),
                                preferred_element_type=jnp.float32)
          + b1_ref[...])
    h1_ref[0] = h1

    @pl.when(jnp.logical_and(b == 0, j == 0))
    def _():
        s1_ref[...] = jnp.zeros_like(s1_ref)
        ss1_ref[...] = jnp.zeros_like(ss1_ref)

    s1_ref[...] += jnp.sum(h1, axis=1, keepdims=True)
    ss1_ref[...] += jnp.sum(h1 * h1, axis=1, keepdims=True)


def _bn_l2_kernel(h1_ref, s1_ref, ss1_ref, g1_ref, be1_ref, W2_ref, b2_ref,
                  h2_ref, s2_ref, ss2_ref, *, count):
    b = pl.program_id(0)
    j = pl.program_id(1)

    mean = s1_ref[...] / count
    var = ss1_ref[...] / count - mean * mean
    rstd = jax.lax.rsqrt(var + 1e-5)
    scale = g1_ref[...] * rstd
    shift = be1_ref[...] - mean * scale

    h1 = jnp.maximum(h1_ref[0] * scale + shift, 0.0)
    h2 = (jax.lax.dot_general(W2_ref[...], h1.astype(jnp.bfloat16),
                              dimension_numbers=(((1,), (0,)), ((), ())),
                              preferred_element_type=jnp.float32)
          + b2_ref[...])
    h2_ref[0] = h2

    @pl.when(jnp.logical_and(b == 0, j == 0))
    def _():
        s2_ref[...] = jnp.zeros_like(s2_ref)
        ss2_ref[...] = jnp.zeros_like(ss2_ref)

    s2_ref[...] += jnp.sum(h2, axis=1, keepdims=True)
    ss2_ref[...] += jnp.sum(h2 * h2, axis=1, keepdims=True)


def _bn_out_kernel(h2_ref, s2_ref, ss2_ref, g2_ref, be2_ref, out_ref, *,
                   count):
    mean = s2_ref[...] / count
    var = ss2_ref[...] / count - mean * mean
    rstd = jax.lax.rsqrt(var + 1e-5)
    scale = g2_ref[...] * rstd
    shift = be2_ref[...] - mean * scale
    out_ref[0] = jnp.maximum(h2_ref[0] * scale + shift, 0.0)


def kernel(p, q, x, y, W1, b1, g1, be1, W2, b2, g2, be2):
    B, N, _ = p.shape
    M = q.shape[1]
    Cx = x.shape[1]
    Cy = y.shape[1]
    C1 = W1.shape[0]
    C2 = W2.shape[0]
    grid = (B, M // BM)
    count = float(B * M)

    pT = jnp.swapaxes(p, 1, 2)
    x_bf = x.astype(jnp.bfloat16)
    y_bf = y.astype(jnp.bfloat16)
    Wx = W1[:, :Cx].astype(jnp.bfloat16)
    Wy = W1[:, Cx:].astype(jnp.bfloat16)
    W2_bf = W2.astype(jnp.bfloat16)
    col = lambda v: v.reshape(-1, 1)

    h1_pre, s1, ss1 = pl.pallas_call(
        functools.partial(_knn_l1_kernel, n_src=N),
        grid=grid,
        in_specs=[
            pl.BlockSpec((1, 3, N), lambda b, j: (b, 0, 0)),
            pl.BlockSpec((1, BM, 3), lambda b, j: (b, j, 0)),
            pl.BlockSpec((1, Cx, N), lambda b, j: (b, 0, 0)),
            pl.BlockSpec((1, Cy, BM), lambda b, j: (b, 0, j)),
            pl.BlockSpec((C1, Cx), lambda b, j: (0, 0)),
            pl.BlockSpec((C1, Cy), lambda b, j: (0, 0)),
            pl.BlockSpec((C1, 1), lambda b, j: (0, 0)),
        ],
        out_specs=[
            pl.BlockSpec((1, C1, BM), lambda b, j: (b, 0, j)),
            pl.BlockSpec((C1, 1), lambda b, j: (0, 0)),
            pl.BlockSpec((C1, 1), lambda b, j: (0, 0)),
        ],
        out_shape=[
            jax.ShapeDtypeStruct((B, C1, M), jnp.float32),
            jax.ShapeDtypeStruct((C1, 1), jnp.float32),
            jax.ShapeDtypeStruct((C1, 1), jnp.float32),
        ],
    )(pT, q, x_bf, y_bf, Wx, Wy, col(b1))

    h2_pre, s2, ss2 = pl.pallas_call(
        functools.partial(_bn_l2_kernel, count=count),
        grid=grid,
        in_specs=[
            pl.BlockSpec((1, C1, BM), lambda b, j: (b, 0, j)),
            pl.BlockSpec((C1, 1), lambda b, j: (0, 0)),
            pl.BlockSpec((C1, 1), lambda b, j: (0, 0)),
            pl.BlockSpec((C1, 1), lambda b, j: (0, 0)),
            pl.BlockSpec((C1, 1), lambda b, j: (0, 0)),
            pl.BlockSpec((C2, C1), lambda b, j: (0, 0)),
            pl.BlockSpec((C2, 1), lambda b, j: (0, 0)),
        ],
        out_specs=[
            pl.BlockSpec((1, C2, BM), lambda b, j: (b, 0, j)),
            pl.BlockSpec((C2, 1), lambda b, j: (0, 0)),
            pl.BlockSpec((C2, 1), lambda b, j: (0, 0)),
        ],
        out_shape=[
            jax.ShapeDtypeStruct((B, C2, M), jnp.float32),
            jax.ShapeDtypeStruct((C2, 1), jnp.float32),
            jax.ShapeDtypeStruct((C2, 1), jnp.float32),
        ],
    )(h1_pre, s1, ss1, col(g1), col(be1), W2_bf, col(b2))

    h = pl.pallas_call(
        functools.partial(_bn_out_kernel, count=count),
        grid=grid,
        in_specs=[
            pl.BlockSpec((1, C2, BM), lambda b, j: (b, 0, j)),
            pl.BlockSpec((C2, 1), lambda b, j: (0, 0)),
            pl.BlockSpec((C2, 1), lambda b, j: (0, 0)),
            pl.BlockSpec((C2, 1), lambda b, j: (0, 0)),
            pl.BlockSpec((C2, 1), lambda b, j: (0, 0)),
        ],
        out_specs=pl.BlockSpec((1, C2, BM), lambda b, j: (b, 0, j)),
        out_shape=jax.ShapeDtypeStruct((B, C2, M), jnp.float32),
    )(h2_pre, s2, ss2, col(g2), col(be2))

    return (q, h)
